# baseline (speedup 1.0000x reference)
import functools

import jax
import jax.numpy as jnp
from jax import lax
from jax.experimental import pallas as pl
from jax.experimental.pallas import tpu as pltpu

_CR = 128

_DN = (((1,), (0,)), ((), ()))


def _body(bm, n_blocks, m_blocks, x_hbm, w1_ref, b1_ref, w2_ref, b2_ref,
          out_ref, xb_ref, stage_ref, sems):
    i = pl.program_id(0)
    n = pl.program_id(1)
    buf = lax.rem(i, 2)
    par = lax.rem(n, 2)
    n_chunks = bm // _CR

    @pl.when((i == 0) & (n == 0))
    def _prologue():
        def start(j, slot):
            pltpu.make_async_copy(
                x_hbm.at[pl.ds(j * _CR, _CR), :], stage_ref.at[slot],
                sems.at[slot]).start()

        start(0, 0)

        def chunk(j, carry):
            slot = lax.rem(j, 2)
            start(j + 1, 1 - slot)
            pltpu.make_async_copy(
                x_hbm.at[pl.ds(j * _CR, _CR), :], stage_ref.at[slot],
                sems.at[slot]).wait()
            xb_ref[lax.div(j, n_chunks), pl.ds(lax.rem(j, n_chunks) * _CR, _CR),
                   :] = stage_ref[slot].astype(jnp.bfloat16)
            return carry

        lax.fori_loop(0, n_chunks + 1, chunk, 0)

    nxt_i = jnp.where(n == n_chunks - 1, i + 1, i)
    nxt_n = jnp.where(n == n_chunks - 1, 0, n + 1)

    @pl.when((n < n_chunks) & (nxt_i < m_blocks - 1)
             & ~((i == 0) & (n == 0)))
    def _start_next():
        pltpu.make_async_copy(
            x_hbm.at[pl.ds((nxt_i + 1) * bm + nxt_n * _CR, _CR), :],
            stage_ref.at[1 - par], sems.at[1 - par]).start()

    def _mm_accum(xb):
        h = lax.dot_general(xb, w1_ref[...], _DN,
                            preferred_element_type=jnp.float32,
                            precision=lax.Precision.DEFAULT)
        h = jnp.maximum(h + b1_ref[...], 0.0).astype(jnp.bfloat16)
        part = lax.dot_general(h, w2_ref[...], _DN,
                               preferred_element_type=jnp.float32,
                               precision=lax.Precision.DEFAULT)
        out_ref[...] = jnp.where(n == 0, part + b2_ref[...],
                                 part + out_ref[...])

    @pl.when(buf == 0)
    def _():
        _mm_accum(xb_ref[0])

    @pl.when(buf == 1)
    def _():
        _mm_accum(xb_ref[1])

    @pl.when((i < m_blocks - 1) & (n < n_chunks) & ~((i == 0) & (n == 0)))
    def _cast_prefetched():
        pltpu.make_async_copy(
            x_hbm.at[pl.ds((i + 1) * bm + n * _CR, _CR), :],
            stage_ref.at[par], sems.at[par]).wait()
        xb_ref[1 - buf, pl.ds(n * _CR, _CR), :] = (
            stage_ref[par].astype(jnp.bfloat16))

    @pl.when(n == n_blocks - 1)
    def _():
        out_ref[...] = jax.nn.sigmoid(out_ref[...])


def _fused_mlp(x, W1, b1, W2, b2, bm, bn):
    m, k = x.shape
    n = W1.shape[1]
    o = W2.shape[1]
    bm = min(bm, m)
    bn = min(bn, n)
    n_blocks = n // bn
    m_blocks = m // bm
    assert n_blocks >= bm // _CR, "chunk events must fit in one m-iteration"
    assert (bm // _CR) % 2 == 0, "even chunk count keeps stage parity aligned"
    body = functools.partial(_body, bm, n_blocks, m_blocks)
    return pl.pallas_call(
        body,
        grid=(m_blocks, n_blocks),
        in_specs=[
            pl.BlockSpec(memory_space=pltpu.MemorySpace.HBM),
            pl.BlockSpec((k, bn), lambda i, j: (0, j)),
            pl.BlockSpec((1, bn), lambda i, j: (0, j)),
            pl.BlockSpec((bn, o), lambda i, j: (j, 0)),
            pl.BlockSpec((1, o), lambda i, j: (0, 0)),
        ],
        out_specs=pl.BlockSpec((bm, o), lambda i, j: (i, 0)),
        out_shape=jax.ShapeDtypeStruct((m, o), jnp.float32),
        scratch_shapes=[
            pltpu.VMEM((2, bm, k), jnp.bfloat16),
            pltpu.VMEM((2, _CR, k), jnp.float32),
            pltpu.SemaphoreType.DMA((2,)),
        ],
        compiler_params=pltpu.CompilerParams(
            dimension_semantics=("arbitrary", "arbitrary"),
            vmem_limit_bytes=67_000_000,
        ),
    )(x, W1, b1.reshape(1, n), W2, b2.reshape(1, o))


def kernel(x, W1, b1, W2, b2):
    return _fused_mlp(x, W1, b1, W2, b2, bm=2048, bn=512)


# speedup vs baseline: 1.0004x; 1.0004x over previous
import functools

import jax
import jax.numpy as jnp
from jax import lax
from jax.experimental import pallas as pl
from jax.experimental.pallas import tpu as pltpu

_CR = 128

_DN = (((1,), (0,)), ((), ()))


def _body(bm, n_blocks, m_blocks, x_hbm, w1_ref, b1_ref, w2_ref, b2_ref,
          out_ref, xb_ref, stage_ref, sems):
    i = pl.program_id(0)
    n = pl.program_id(1)
    buf = lax.rem(i, 2)
    par = lax.rem(n, 2)
    n_chunks = bm // _CR

    @pl.when((i == 0) & (n == 0))
    def _prologue():
        def start(j, slot):
            pltpu.make_async_copy(
                x_hbm.at[pl.ds(j * _CR, _CR), :], stage_ref.at[slot],
                sems.at[slot]).start()

        start(0, 0)

        def chunk(j, carry):
            slot = lax.rem(j, 2)
            start(j + 1, 1 - slot)
            pltpu.make_async_copy(
                x_hbm.at[pl.ds(j * _CR, _CR), :], stage_ref.at[slot],
                sems.at[slot]).wait()
            xb_ref[lax.div(j, n_chunks), pl.ds(lax.rem(j, n_chunks) * _CR, _CR),
                   :] = stage_ref[slot].astype(jnp.bfloat16)
            return carry

        lax.fori_loop(0, n_chunks + 1, chunk, 0)

    nxt_i = jnp.where(n == n_chunks - 1, i + 1, i)
    nxt_n = jnp.where(n == n_chunks - 1, 0, n + 1)

    @pl.when((n < n_chunks) & (nxt_i < m_blocks - 1)
             & ~((i == 0) & (n == 0)))
    def _start_next():
        pltpu.make_async_copy(
            x_hbm.at[pl.ds((nxt_i + 1) * bm + nxt_n * _CR, _CR), :],
            stage_ref.at[1 - par], sems.at[1 - par]).start()

    def _mm_accum(xb):
        h = lax.dot_general(xb, w1_ref[...], _DN,
                            preferred_element_type=jnp.float32,
                            precision=lax.Precision.DEFAULT)
        h = jnp.maximum(h + b1_ref[...], 0.0)
        part = lax.dot_general(h, w2_ref[...], _DN,
                               preferred_element_type=jnp.float32,
                               precision=lax.Precision.DEFAULT)
        out_ref[...] = jnp.where(n == 0, part + b2_ref[...],
                                 part + out_ref[...])

    @pl.when(buf == 0)
    def _():
        _mm_accum(xb_ref[0])

    @pl.when(buf == 1)
    def _():
        _mm_accum(xb_ref[1])

    @pl.when((i < m_blocks - 1) & (n < n_chunks) & ~((i == 0) & (n == 0)))
    def _cast_prefetched():
        pltpu.make_async_copy(
            x_hbm.at[pl.ds((i + 1) * bm + n * _CR, _CR), :],
            stage_ref.at[par], sems.at[par]).wait()
        xb_ref[1 - buf, pl.ds(n * _CR, _CR), :] = (
            stage_ref[par].astype(jnp.bfloat16))

    @pl.when(n == n_blocks - 1)
    def _():
        out_ref[...] = jax.nn.sigmoid(out_ref[...])


def _fused_mlp(x, W1, b1, W2, b2, bm, bn):
    m, k = x.shape
    n = W1.shape[1]
    o = W2.shape[1]
    bm = min(bm, m)
    bn = min(bn, n)
    n_blocks = n // bn
    m_blocks = m // bm
    assert n_blocks >= bm // _CR, "chunk events must fit in one m-iteration"
    assert (bm // _CR) % 2 == 0, "even chunk count keeps stage parity aligned"
    body = functools.partial(_body, bm, n_blocks, m_blocks)
    return pl.pallas_call(
        body,
        grid=(m_blocks, n_blocks),
        in_specs=[
            pl.BlockSpec(memory_space=pltpu.MemorySpace.HBM),
            pl.BlockSpec((k, bn), lambda i, j: (0, j)),
            pl.BlockSpec((1, bn), lambda i, j: (0, j)),
            pl.BlockSpec((bn, o), lambda i, j: (j, 0)),
            pl.BlockSpec((1, o), lambda i, j: (0, 0)),
        ],
        out_specs=pl.BlockSpec((bm, o), lambda i, j: (i, 0)),
        out_shape=jax.ShapeDtypeStruct((m, o), jnp.float32),
        scratch_shapes=[
            pltpu.VMEM((2, bm, k), jnp.bfloat16),
            pltpu.VMEM((2, _CR, k), jnp.float32),
            pltpu.SemaphoreType.DMA((2,)),
        ],
        compiler_params=pltpu.CompilerParams(
            dimension_semantics=("arbitrary", "arbitrary"),
            vmem_limit_bytes=67_000_000,
        ),
    )(x, W1, b1.reshape(1, n), W2, b2.reshape(1, o))


def kernel(x, W1, b1, W2, b2):
    return _fused_mlp(x, W1, b1, W2, b2, bm=2048, bn=512)
